# baseline (speedup 1.0000x reference)
import jax
import jax.numpy as jnp
from jax import lax
from jax.experimental import pallas as pl
from jax.experimental.pallas import tpu as pltpu
from jax.experimental.pallas import tpu_sc as plsc

_F32 = jnp.float32
_I32 = jnp.int32


def _front_body(x_ref, w_ref, wcat_ref, b_ref, h_ref, s_ref):
    n = x_ref.shape[0]
    h0 = jnp.dot(x_ref[...], w_ref[...], preferred_element_type=_F32,
                 precision=lax.Precision.HIGHEST)
    mean = jnp.sum(h0, axis=0, keepdims=True) * (1.0 / n)
    xc = h0 - mean
    var = jnp.sum(xc * xc, axis=0, keepdims=True) * (1.0 / n)
    hn = xc * lax.rsqrt(var + 1e-5)
    h_ref[...] = hn
    s_ref[...] = jnp.dot(hn, wcat_ref[...], preferred_element_type=_F32,
                         precision=lax.Precision.HIGHEST) + b_ref[...]


def _add_body(p_ref, o_ref):
    o_ref[...] = p_ref[0] + p_ref[1]


def _make_sc_edge(N, Npad, E, C):
    NS = 16
    NC = 2
    G = 16
    NCHUNK = Npad // NS
    EP = E // NS
    SB = 80
    BL = 2000
    NBL = EP // BL
    NSB = BL // SB
    NPIPE = NSB // 2
    GB = SB // G
    EP2 = EP // NC
    NBL2 = EP2 // BL
    CW = C // G
    assert EP * NS == E and BL * NBL == EP and SB * NSB == BL
    assert EP2 * NC == EP and BL * NBL2 == EP2 and CW * G == C
    assert NCHUNK % SB == 0 and NBL == NC * NBL2 and NSB == 2 * NPIPE + 1

    mesh = plsc.VectorSubcoreMesh(core_axis_name="c", subcore_axis_name="s")

    def body(row_hbm, col_hbm, attr_hbm, s1_hbm, s2_hbm, h_hbm,
             alpha_hbm, part_hbm,
             tbl_a, tbl_b, rows_b, cols_b, av_b, dchunk,
             nrm0, nrm1, ridx0, ridx1, cidx0, cidx1, vblk0, vblk1,
             rowbuf0, rowbuf1, gsem0, gsem1, ssem0, ssem1,
             deg_sh, acc_sh):
        cid = lax.axis_index("c")
        sid = lax.axis_index("s")
        ebase = sid * EP
        zv = jnp.zeros((G,), _F32)
        slots = ((ridx0, cidx0, vblk0, nrm0, rowbuf0, gsem0, ssem0),
                 (ridx1, cidx1, vblk1, nrm1, rowbuf1, gsem1, ssem1))

        def _zrow(i, c):
            for j in range(CW):
                rowbuf0[i, pl.ds(j * G, G)] = zv
            return c
        lax.fori_loop(0, SB, _zrow, 0)

        def _zd(i, c):
            dchunk[pl.ds(i * G, G)] = zv
            return c
        lax.fori_loop(0, SB // G, _zd, 0)
        for k in range(NCHUNK // SB):
            pltpu.sync_copy(rowbuf0, acc_sh.at[pl.ds(sid * NCHUNK + k * SB, SB)])
            pltpu.sync_copy(dchunk, deg_sh.at[pl.ds(sid * NCHUNK + k * SB, SB)])
        plsc.subcore_barrier()

        sc1 = jax.named_scope("p1_alpha_deg")
        sc1.__enter__()
        pltpu.sync_copy(s1_hbm, tbl_a.at[pl.ds(0, N)])
        pltpu.sync_copy(s2_hbm, tbl_b)
        one = jnp.ones((G,), _F32)

        def _stage1(j, ridx, vblk):
            for gg in range(GB):
                boff = j * SB + gg * G
                r = rows_b[pl.ds(boff, G)]
                cc = cols_b[pl.ds(boff, G)]
                ea = av_b[pl.ds(boff, G)]
                z = plsc.load_gather(tbl_a, [r]) + plsc.load_gather(tbl_b, [cc])
                sig = 1.0 / (1.0 + jnp.exp(-z))
                a = jnp.where(r == cc, one, sig * ea)
                av_b[pl.ds(boff, G)] = a
                ridx[0, pl.ds(gg * G, G)] = r
                vblk[0, pl.ds(gg * G, G)] = jnp.abs(a)

        def _p1blk(blk, c):
            eoff = ebase + blk * BL
            pltpu.sync_copy(row_hbm.at[pl.ds(eoff, BL)], rows_b)
            pltpu.sync_copy(col_hbm.at[pl.ds(eoff, BL)], cols_b)
            pltpu.sync_copy(attr_hbm.at[pl.ds(eoff, BL)], av_b)

            def _pipe1(t, c2):
                for p in range(2):
                    ridx, _, vblk, _, _, gsem, _ = slots[p]

                    @pl.when(t > 0)
                    def _():
                        pltpu.make_async_copy(
                            vblk.at[0], deg_sh.at[ridx.at[0]], gsem).wait()
                    _stage1(2 * t + p, ridx, vblk)
                    pltpu.async_copy(vblk.at[0], deg_sh.at[ridx.at[0]],
                                     gsem, add=True)
                return c2
            lax.fori_loop(0, NPIPE, _pipe1, 0)
            for p in range(2):
                ridx, _, vblk, _, _, gsem, _ = slots[p]
                pltpu.make_async_copy(vblk.at[0], deg_sh.at[ridx.at[0]],
                                      gsem).wait()
            _stage1(NSB - 1, ridx0, vblk0)
            pltpu.sync_copy(vblk0.at[0], deg_sh.at[ridx0.at[0]], add=True)

            own0 = jnp.logical_and(cid == 0, blk < NBL2)
            own1 = jnp.logical_and(cid == 1, blk >= NBL2)

            @pl.when(jnp.logical_or(own0, own1))
            def _():
                pltpu.sync_copy(av_b, alpha_hbm.at[pl.ds(eoff, BL)])
            return c
        lax.fori_loop(0, NBL, _p1blk, 0)
        plsc.subcore_barrier()
        sc1.__exit__(None, None, None)

        sc2 = jax.named_scope("p15_dinv")
        sc2.__enter__()
        th = jnp.full((G,), 1.5, _F32)
        hf = jnp.full((G,), 0.5, _F32)
        magic = jnp.full((G,), 0x5F3759DF, _I32)

        def _dk(k, c):
            doff = sid * NCHUNK + k * SB
            pltpu.sync_copy(deg_sh.at[pl.ds(doff, SB)], dchunk)
            for g in range(GB):
                d = dchunk[pl.ds(g * G, G)]
                y = plsc.bitcast(magic - (plsc.bitcast(d, _I32) >> 1), _F32)
                hd = hf * d
                y = y * (th - hd * y * y)
                y = y * (th - hd * y * y)
                y = y * (th - hd * y * y)
                dchunk[pl.ds(g * G, G)] = jnp.where(d > 0.0, y, zv)
            pltpu.sync_copy(dchunk, deg_sh.at[pl.ds(doff, SB)])
            return c
        lax.fori_loop(0, NCHUNK // SB, _dk, 0)
        plsc.subcore_barrier()
        pltpu.sync_copy(deg_sh, tbl_a)
        sc2.__exit__(None, None, None)

        sc3 = jax.named_scope("p2_aggregate")
        sc3.__enter__()
        def _stage2(j, ridx, cidx, nrm):
            for gg in range(GB):
                boff = j * SB + gg * G
                r = rows_b[pl.ds(boff, G)]
                cc = cols_b[pl.ds(boff, G)]
                a = av_b[pl.ds(boff, G)]
                d1 = plsc.load_gather(tbl_a, [r])
                d2 = plsc.load_gather(tbl_a, [cc])
                ridx[0, pl.ds(gg * G, G)] = r
                cidx[0, pl.ds(gg * G, G)] = cc
                nrm[pl.ds(gg * G, G)] = d1 * a * d2

        def _scale(rowbuf, nrm):
            def _se(e, c):
                nb = plsc.load_gather(nrm, [jnp.full((G,), 0, _I32) + e])
                for jj in range(CW):
                    rowbuf[e, pl.ds(jj * G, G)] = rowbuf[e, pl.ds(jj * G, G)] * nb
                return c
            lax.fori_loop(0, SB, _se, 0)

        def _p2blk(blk, c):
            eoff = ebase + cid * EP2 + blk * BL
            pltpu.sync_copy(row_hbm.at[pl.ds(eoff, BL)], rows_b)
            pltpu.sync_copy(col_hbm.at[pl.ds(eoff, BL)], cols_b)
            pltpu.sync_copy(alpha_hbm.at[pl.ds(eoff, BL)], av_b)

            def _pipe2(t, c2):
                for p in range(2):
                    ridx, cidx, _, nrm, rowbuf, gsem, ssem = slots[p]

                    @pl.when(t > 0)
                    def _():
                        pltpu.make_async_copy(
                            rowbuf, acc_sh.at[ridx.at[0]], ssem).wait()
                    _stage2(2 * t + p, ridx, cidx, nrm)
                    pltpu.async_copy(h_hbm.at[cidx.at[0]], rowbuf, gsem)
                for p in range(2):
                    ridx, cidx, _, nrm, rowbuf, gsem, ssem = slots[p]
                    pltpu.make_async_copy(h_hbm.at[cidx.at[0]], rowbuf,
                                          gsem).wait()
                    _scale(rowbuf, nrm)
                    pltpu.async_copy(rowbuf, acc_sh.at[ridx.at[0]],
                                     ssem, add=True)
                return c2
            lax.fori_loop(0, NPIPE, _pipe2, 0)
            for p in range(2):
                ridx, _, _, _, rowbuf, _, ssem = slots[p]
                pltpu.make_async_copy(rowbuf, acc_sh.at[ridx.at[0]],
                                      ssem).wait()
            _stage2(NSB - 1, ridx0, cidx0, nrm0)
            pltpu.sync_copy(h_hbm.at[cidx0.at[0]], rowbuf0)
            _scale(rowbuf0, nrm0)
            pltpu.sync_copy(rowbuf0, acc_sh.at[ridx0.at[0]], add=True)
            return c
        lax.fori_loop(0, NBL2, _p2blk, 0)
        plsc.subcore_barrier()
        sc3.__exit__(None, None, None)

        for k in range(NCHUNK // SB):
            roff = sid * NCHUNK + k * SB
            pltpu.sync_copy(acc_sh.at[pl.ds(roff, SB)], rowbuf0)
            pltpu.sync_copy(rowbuf0, part_hbm.at[cid, pl.ds(roff, SB)])

    return pl.kernel(
        body,
        out_type=(jax.ShapeDtypeStruct((E,), _F32),
                  jax.ShapeDtypeStruct((NC, Npad, C), _F32)),
        mesh=mesh,
        compiler_params=pltpu.CompilerParams(needs_layout_passes=False,
                                             use_tc_tiling_on_sc=False),
        scratch_types=[
            pltpu.VMEM((Npad,), _F32),
            pltpu.VMEM((N,), _F32),
            pltpu.VMEM((BL,), _I32),
            pltpu.VMEM((BL,), _I32),
            pltpu.VMEM((BL,), _F32),
            pltpu.VMEM((SB,), _F32),
            pltpu.VMEM((SB,), _F32),
            pltpu.VMEM((SB,), _F32),
            pltpu.VMEM((1, SB), _I32),
            pltpu.VMEM((1, SB), _I32),
            pltpu.VMEM((1, SB), _I32),
            pltpu.VMEM((1, SB), _I32),
            pltpu.VMEM((1, SB), _F32),
            pltpu.VMEM((1, SB), _F32),
            pltpu.VMEM((SB, C), _F32),
            pltpu.VMEM((SB, C), _F32),
            pltpu.SemaphoreType.DMA,
            pltpu.SemaphoreType.DMA,
            pltpu.SemaphoreType.DMA,
            pltpu.SemaphoreType.DMA,
            pltpu.VMEM_SHARED((Npad,), _F32),
            pltpu.VMEM_SHARED((Npad, C), _F32),
        ],
    )


def kernel(x, edge_index, edge_attr, batch_mask, weight, alpha_fc_w, alpha_fc_b):
    N, IC = x.shape
    OC = weight.shape[1]
    E = edge_index.shape[1]
    Npad = ((N + 1279) // 1280) * 1280

    row = edge_index[0].astype(_I32)
    col = edge_index[1].astype(_I32)
    attr = edge_attr.astype(_F32)

    w1 = alpha_fc_w[0, :OC]
    w2 = alpha_fc_w[0, OC:]
    wcat = jnp.zeros((OC, 128), _F32)
    wcat = wcat.at[:, 0].set(w1).at[:, 1].set(w2)
    bvec = jnp.zeros((1, 128), _F32).at[0, 0].set(alpha_fc_b[0])

    h, s = pl.pallas_call(
        _front_body,
        out_shape=(jax.ShapeDtypeStruct((N, OC), _F32),
                   jax.ShapeDtypeStruct((N, 128), _F32)),
    )(x, weight, wcat, bvec)
    s1 = s[:, 0]
    s2 = s[:, 1]

    sc_edge = _make_sc_edge(N, Npad, E, OC)
    alpha, partials = sc_edge(row, col, attr, s1, s2, h)

    CH = Npad // 8
    out2 = pl.pallas_call(
        _add_body,
        grid=(8,),
        in_specs=[pl.BlockSpec((2, CH, OC), lambda i: (0, i, 0))],
        out_specs=pl.BlockSpec((CH, OC), lambda i: (i, 0)),
        out_shape=jax.ShapeDtypeStruct((Npad, OC), _F32),
    )(partials)

    out = out2[:N].reshape(N, OC, 1)
    return (out, alpha.reshape(E, 1), edge_index)


# speedup vs baseline: 1.1709x; 1.1709x over previous
import jax
import jax.numpy as jnp
from jax import lax
from jax.experimental import pallas as pl
from jax.experimental.pallas import tpu as pltpu
from jax.experimental.pallas import tpu_sc as plsc

_F32 = jnp.float32
_I32 = jnp.int32


def _front_body(x_ref, w_ref, wpair_ref, b_ref, h_ref, s_ref):
    n = x_ref.shape[0]
    h0 = jnp.dot(x_ref[...], w_ref[...], preferred_element_type=_F32,
                 precision=lax.Precision.HIGHEST)
    mean = jnp.sum(h0, axis=0, keepdims=True) * (1.0 / n)
    xc = h0 - mean
    var = jnp.sum(xc * xc, axis=0, keepdims=True) * (1.0 / n)
    hn = xc * lax.rsqrt(var + 1e-5)
    h_ref[...] = hn
    s_ref[...] = lax.dot_general(
        wpair_ref[...], hn, (((1,), (1,)), ((), ())),
        preferred_element_type=_F32,
        precision=lax.Precision.HIGHEST) + b_ref[...]


def _add_body(p_ref, o_ref):
    o_ref[...] = p_ref[0] + p_ref[1]


def _make_sc_edge(N, Npad, E, C):
    NS = 16
    NC = 2
    G = 16
    NCHUNK = Npad // NS
    EP = E // NS
    SB = 80
    BL = 2000
    NBL = EP // BL
    NSB = BL // SB
    NPIPE = NSB // 2
    GB = SB // G
    EP2 = EP // NC
    NBL2 = EP2 // BL
    CW = C // G
    assert EP * NS == E and BL * NBL == EP and SB * NSB == BL
    assert EP2 * NC == EP and BL * NBL2 == EP2 and CW * G == C
    assert NCHUNK % SB == 0 and NBL == NC * NBL2 and NSB == 2 * NPIPE + 1

    mesh = plsc.VectorSubcoreMesh(core_axis_name="c", subcore_axis_name="s")

    def body(row_hbm, col_hbm, attr_hbm, s1_hbm, s2_hbm, h_hbm,
             alpha_hbm, part_hbm,
             tbl_a, tbl_b, rows_b, cols_b, av_b, dchunk,
             nrm0, nrm1, ridx0, ridx1, cidx0, cidx1, vblk0, vblk1,
             rowbuf0, rowbuf1, gsem0, gsem1, ssem0, ssem1,
             deg_sh, acc_sh):
        cid = lax.axis_index("c")
        sid = lax.axis_index("s")
        ebase = sid * EP
        zv = jnp.zeros((G,), _F32)
        slots = ((ridx0, cidx0, vblk0, nrm0, rowbuf0, gsem0, ssem0),
                 (ridx1, cidx1, vblk1, nrm1, rowbuf1, gsem1, ssem1))

        def _zrow(i, c):
            for j in range(CW):
                rowbuf0[i, pl.ds(j * G, G)] = zv
            return c
        lax.fori_loop(0, SB, _zrow, 0)

        def _zd(i, c):
            dchunk[pl.ds(i * G, G)] = zv
            return c
        lax.fori_loop(0, SB // G, _zd, 0)
        for k in range(NCHUNK // SB):
            pltpu.sync_copy(rowbuf0, acc_sh.at[pl.ds(sid * NCHUNK + k * SB, SB)])
            pltpu.sync_copy(dchunk, deg_sh.at[pl.ds(sid * NCHUNK + k * SB, SB)])
        plsc.subcore_barrier()

        sc1 = jax.named_scope("p1_alpha_deg")
        sc1.__enter__()
        pltpu.sync_copy(s1_hbm, tbl_a.at[pl.ds(0, N)])
        pltpu.sync_copy(s2_hbm, tbl_b)
        one = jnp.ones((G,), _F32)

        def _stage1(j, ridx, vblk):
            for gg in range(GB):
                boff = j * SB + gg * G
                r = rows_b[pl.ds(boff, G)]
                cc = cols_b[pl.ds(boff, G)]
                ea = av_b[pl.ds(boff, G)]
                z = plsc.load_gather(tbl_a, [r]) + plsc.load_gather(tbl_b, [cc])
                sig = 1.0 / (1.0 + jnp.exp(-z))
                a = jnp.where(r == cc, one, sig * ea)
                av_b[pl.ds(boff, G)] = a
                ridx[0, pl.ds(gg * G, G)] = r
                vblk[0, pl.ds(gg * G, G)] = jnp.abs(a)

        def _p1blk(blk, c):
            eoff = ebase + blk * BL
            pltpu.sync_copy(row_hbm.at[pl.ds(eoff, BL)], rows_b)
            pltpu.sync_copy(col_hbm.at[pl.ds(eoff, BL)], cols_b)
            pltpu.sync_copy(attr_hbm.at[pl.ds(eoff, BL)], av_b)

            def _pipe1(t, c2):
                for p in range(2):
                    ridx, _, vblk, _, _, gsem, _ = slots[p]

                    @pl.when(t > 0)
                    def _():
                        pltpu.make_async_copy(
                            vblk.at[0], deg_sh.at[ridx.at[0]], gsem).wait()
                    _stage1(2 * t + p, ridx, vblk)
                    pltpu.async_copy(vblk.at[0], deg_sh.at[ridx.at[0]],
                                     gsem, add=True)
                return c2
            lax.fori_loop(0, NPIPE, _pipe1, 0)
            for p in range(2):
                ridx, _, vblk, _, _, gsem, _ = slots[p]
                pltpu.make_async_copy(vblk.at[0], deg_sh.at[ridx.at[0]],
                                      gsem).wait()
            _stage1(NSB - 1, ridx0, vblk0)
            pltpu.sync_copy(vblk0.at[0], deg_sh.at[ridx0.at[0]], add=True)

            own0 = jnp.logical_and(cid == 0, blk < NBL2)
            own1 = jnp.logical_and(cid == 1, blk >= NBL2)

            @pl.when(jnp.logical_or(own0, own1))
            def _():
                pltpu.sync_copy(av_b, alpha_hbm.at[pl.ds(eoff, BL)])
            return c
        lax.fori_loop(0, NBL, _p1blk, 0)
        plsc.subcore_barrier()
        sc1.__exit__(None, None, None)

        sc2 = jax.named_scope("p15_dinv")
        sc2.__enter__()
        th = jnp.full((G,), 1.5, _F32)
        hf = jnp.full((G,), 0.5, _F32)
        magic = jnp.full((G,), 0x5F3759DF, _I32)

        def _dk(k, c):
            doff = sid * NCHUNK + k * SB
            pltpu.sync_copy(deg_sh.at[pl.ds(doff, SB)], dchunk)
            for g in range(GB):
                d = dchunk[pl.ds(g * G, G)]
                y = plsc.bitcast(magic - (plsc.bitcast(d, _I32) >> 1), _F32)
                hd = hf * d
                y = y * (th - hd * y * y)
                y = y * (th - hd * y * y)
                y = y * (th - hd * y * y)
                dchunk[pl.ds(g * G, G)] = jnp.where(d > 0.0, y, zv)
            pltpu.sync_copy(dchunk, deg_sh.at[pl.ds(doff, SB)])
            return c
        lax.fori_loop(0, NCHUNK // SB, _dk, 0)
        plsc.subcore_barrier()
        pltpu.sync_copy(deg_sh, tbl_a)
        sc2.__exit__(None, None, None)

        sc3 = jax.named_scope("p2_aggregate")
        sc3.__enter__()
        def _stage2(j, ridx, cidx, nrm):
            for gg in range(GB):
                boff = j * SB + gg * G
                r = rows_b[pl.ds(boff, G)]
                cc = cols_b[pl.ds(boff, G)]
                a = av_b[pl.ds(boff, G)]
                d1 = plsc.load_gather(tbl_a, [r])
                d2 = plsc.load_gather(tbl_a, [cc])
                ridx[0, pl.ds(gg * G, G)] = r
                cidx[0, pl.ds(gg * G, G)] = cc
                nrm[pl.ds(gg * G, G)] = d1 * a * d2

        def _scale(rowbuf, nrm):
            def _sg(g, c):
                nvec = nrm[pl.ds(g * G, G)]
                for e in range(G):
                    nb = nvec.at[jnp.full((G,), e, _I32)].get(
                        mode="promise_in_bounds")
                    eidx = g * G + e
                    for jj in range(CW):
                        rowbuf[eidx, pl.ds(jj * G, G)] = (
                            rowbuf[eidx, pl.ds(jj * G, G)] * nb)
                return c
            lax.fori_loop(0, GB, _sg, 0)

        def _p2blk(blk, c):
            eoff = ebase + cid * EP2 + blk * BL
            pltpu.sync_copy(row_hbm.at[pl.ds(eoff, BL)], rows_b)
            pltpu.sync_copy(col_hbm.at[pl.ds(eoff, BL)], cols_b)
            pltpu.sync_copy(alpha_hbm.at[pl.ds(eoff, BL)], av_b)

            def _pipe2(t, c2):
                for p in range(2):
                    ridx, cidx, _, nrm, rowbuf, gsem, ssem = slots[p]

                    @pl.when(t > 0)
                    def _():
                        pltpu.make_async_copy(
                            rowbuf, acc_sh.at[ridx.at[0]], ssem).wait()
                    _stage2(2 * t + p, ridx, cidx, nrm)
                    pltpu.async_copy(h_hbm.at[cidx.at[0]], rowbuf, gsem)
                for p in range(2):
                    ridx, cidx, _, nrm, rowbuf, gsem, ssem = slots[p]
                    pltpu.make_async_copy(h_hbm.at[cidx.at[0]], rowbuf,
                                          gsem).wait()
                    _scale(rowbuf, nrm)
                    pltpu.async_copy(rowbuf, acc_sh.at[ridx.at[0]],
                                     ssem, add=True)
                return c2
            lax.fori_loop(0, NPIPE, _pipe2, 0)
            for p in range(2):
                ridx, _, _, _, rowbuf, _, ssem = slots[p]
                pltpu.make_async_copy(rowbuf, acc_sh.at[ridx.at[0]],
                                      ssem).wait()
            _stage2(NSB - 1, ridx0, cidx0, nrm0)
            pltpu.sync_copy(h_hbm.at[cidx0.at[0]], rowbuf0)
            _scale(rowbuf0, nrm0)
            pltpu.sync_copy(rowbuf0, acc_sh.at[ridx0.at[0]], add=True)
            return c
        lax.fori_loop(0, NBL2, _p2blk, 0)
        plsc.subcore_barrier()
        sc3.__exit__(None, None, None)

        for k in range(NCHUNK // SB):
            roff = sid * NCHUNK + k * SB
            pltpu.sync_copy(acc_sh.at[pl.ds(roff, SB)], rowbuf0)
            pltpu.sync_copy(rowbuf0, part_hbm.at[cid, pl.ds(roff, SB)])

    return pl.kernel(
        body,
        out_type=(jax.ShapeDtypeStruct((E,), _F32),
                  jax.ShapeDtypeStruct((NC, Npad, C), _F32)),
        mesh=mesh,
        compiler_params=pltpu.CompilerParams(needs_layout_passes=False,
                                             use_tc_tiling_on_sc=False),
        scratch_types=[
            pltpu.VMEM((Npad,), _F32),
            pltpu.VMEM((N,), _F32),
            pltpu.VMEM((BL,), _I32),
            pltpu.VMEM((BL,), _I32),
            pltpu.VMEM((BL,), _F32),
            pltpu.VMEM((SB,), _F32),
            pltpu.VMEM((SB,), _F32),
            pltpu.VMEM((SB,), _F32),
            pltpu.VMEM((1, SB), _I32),
            pltpu.VMEM((1, SB), _I32),
            pltpu.VMEM((1, SB), _I32),
            pltpu.VMEM((1, SB), _I32),
            pltpu.VMEM((1, SB), _F32),
            pltpu.VMEM((1, SB), _F32),
            pltpu.VMEM((SB, C), _F32),
            pltpu.VMEM((SB, C), _F32),
            pltpu.SemaphoreType.DMA,
            pltpu.SemaphoreType.DMA,
            pltpu.SemaphoreType.DMA,
            pltpu.SemaphoreType.DMA,
            pltpu.VMEM_SHARED((Npad,), _F32),
            pltpu.VMEM_SHARED((Npad, C), _F32),
        ],
    )


def kernel(x, edge_index, edge_attr, batch_mask, weight, alpha_fc_w, alpha_fc_b):
    N, IC = x.shape
    OC = weight.shape[1]
    E = edge_index.shape[1]
    Npad = ((N + 1279) // 1280) * 1280

    row = edge_index[0].astype(_I32)
    col = edge_index[1].astype(_I32)
    attr = edge_attr.astype(_F32)

    wpair = alpha_fc_w[0].reshape(2, OC)
    bpair = jnp.zeros((2, 1), _F32).at[0, 0].set(alpha_fc_b[0])

    h, s = pl.pallas_call(
        _front_body,
        out_shape=(jax.ShapeDtypeStruct((N, OC), _F32),
                   jax.ShapeDtypeStruct((2, N), _F32)),
    )(x, weight, wpair, bpair)
    s1 = s[0]
    s2 = s[1]

    sc_edge = _make_sc_edge(N, Npad, E, OC)
    alpha, partials = sc_edge(row, col, attr, s1, s2, h)

    CH = N // 5
    out2 = pl.pallas_call(
        _add_body,
        grid=(5,),
        in_specs=[pl.BlockSpec((2, CH, OC), lambda i: (0, i, 0))],
        out_specs=pl.BlockSpec((CH, OC), lambda i: (i, 0)),
        out_shape=jax.ShapeDtypeStruct((N, OC), _F32),
    )(partials)

    out = out2.reshape(N, OC, 1)
    return (out, alpha.reshape(E, 1), edge_index)


# speedup vs baseline: 1.2098x; 1.0333x over previous
import jax
import jax.numpy as jnp
from jax import lax
from jax.experimental import pallas as pl
from jax.experimental.pallas import tpu as pltpu
from jax.experimental.pallas import tpu_sc as plsc

_F32 = jnp.float32
_I32 = jnp.int32


def _front_body(x_ref, w_ref, wpair_ref, b_ref, h_ref, s_ref):
    n = x_ref.shape[0]
    h0 = jnp.dot(x_ref[...], w_ref[...], preferred_element_type=_F32)
    mean = jnp.sum(h0, axis=0, keepdims=True) * (1.0 / n)
    xc = h0 - mean
    var = jnp.sum(xc * xc, axis=0, keepdims=True) * (1.0 / n)
    hn = xc * lax.rsqrt(var + 1e-5)
    h_ref[...] = hn
    s_ref[...] = lax.dot_general(
        wpair_ref[...], hn, (((1,), (1,)), ((), ())),
        preferred_element_type=_F32,
        precision=lax.Precision.HIGHEST) + b_ref[...]


def _add_body(p_ref, o_ref):
    o_ref[...] = p_ref[0] + p_ref[1]


def _make_sc_edge(N, Npad, E, C):
    NS = 16
    NC = 2
    G = 16
    NCHUNK = Npad // NS
    EP = E // NS
    SB = 80
    BL = 2000
    NBL = EP // BL
    NSB = BL // SB
    NPIPE = NSB // 2
    GB = SB // G
    EP2 = EP // NC
    NBL2 = EP2 // BL
    CW = C // G
    assert EP * NS == E and BL * NBL == EP and SB * NSB == BL
    assert EP2 * NC == EP and BL * NBL2 == EP2 and CW * G == C
    assert NCHUNK % SB == 0 and NBL == NC * NBL2 and NSB == 2 * NPIPE + 1

    mesh = plsc.VectorSubcoreMesh(core_axis_name="c", subcore_axis_name="s")

    def body(row_hbm, col_hbm, attr_hbm, s1_hbm, s2_hbm, h_hbm,
             alpha_hbm, part_hbm,
             tbl_a, tbl_b, rows_b, cols_b, av_b, dchunk,
             nrm0, nrm1, ridx0, ridx1, cidx0, cidx1,
             vblk0, vblk1, vblk2, vblk3,
             rowbuf0, rowbuf1, gsem0, gsem1, ssem0, ssem1,
             deg_sh, acc_sh):
        cid = lax.axis_index("c")
        sid = lax.axis_index("s")
        ebase = sid * EP
        zv = jnp.zeros((G,), _F32)
        slots = ((ridx0, cidx0, vblk0, nrm0, rowbuf0, gsem0, ssem0),
                 (ridx1, cidx1, vblk1, nrm1, rowbuf1, gsem1, ssem1))
        dslots = ((ridx0, vblk0, gsem0), (ridx1, vblk1, gsem1),
                  (cidx0, vblk2, ssem0), (cidx1, vblk3, ssem1))

        def _zrow(i, c):
            for j in range(CW):
                rowbuf0[i, pl.ds(j * G, G)] = zv
            return c
        lax.fori_loop(0, SB, _zrow, 0)

        def _zd(i, c):
            dchunk[pl.ds(i * G, G)] = zv
            return c
        lax.fori_loop(0, SB // G, _zd, 0)
        for k in range(NCHUNK // SB):
            pltpu.sync_copy(rowbuf0, acc_sh.at[pl.ds(sid * NCHUNK + k * SB, SB)])
            pltpu.sync_copy(dchunk, deg_sh.at[pl.ds(sid * NCHUNK + k * SB, SB)])
        plsc.subcore_barrier()

        sc1 = jax.named_scope("p1_alpha_deg")
        sc1.__enter__()
        pltpu.sync_copy(s1_hbm, tbl_a.at[pl.ds(0, N)])
        pltpu.sync_copy(s2_hbm, tbl_b)
        one = jnp.ones((G,), _F32)

        def _stage1(j, ridx, vblk):
            for gg in range(GB):
                boff = j * SB + gg * G
                r = rows_b[pl.ds(boff, G)]
                cc = cols_b[pl.ds(boff, G)]
                ea = av_b[pl.ds(boff, G)]
                z = plsc.load_gather(tbl_a, [r]) + plsc.load_gather(tbl_b, [cc])
                sig = 1.0 / (1.0 + jnp.exp(-z))
                a = jnp.where(r == cc, one, sig * ea)
                av_b[pl.ds(boff, G)] = a
                ridx[0, pl.ds(gg * G, G)] = r
                vblk[0, pl.ds(gg * G, G)] = jnp.abs(a)

        def _p1blk(blk, c):
            eoff = ebase + blk * BL
            pltpu.sync_copy(row_hbm.at[pl.ds(eoff, BL)], rows_b)
            pltpu.sync_copy(col_hbm.at[pl.ds(eoff, BL)], cols_b)
            pltpu.sync_copy(attr_hbm.at[pl.ds(eoff, BL)], av_b)

            def _pipe1(t, c2):
                for p in range(4):
                    ridx, vblk, gsem = dslots[p]

                    @pl.when(t > 0)
                    def _():
                        pltpu.make_async_copy(
                            vblk.at[0], deg_sh.at[ridx.at[0]], gsem).wait()
                    _stage1(4 * t + p, ridx, vblk)
                    pltpu.async_copy(vblk.at[0], deg_sh.at[ridx.at[0]],
                                     gsem, add=True)
                return c2
            lax.fori_loop(0, NSB // 4, _pipe1, 0)
            for p in range(4):
                ridx, vblk, gsem = dslots[p]
                pltpu.make_async_copy(vblk.at[0], deg_sh.at[ridx.at[0]],
                                      gsem).wait()
            _stage1(NSB - 1, ridx0, vblk0)
            pltpu.sync_copy(vblk0.at[0], deg_sh.at[ridx0.at[0]], add=True)

            own0 = jnp.logical_and(cid == 0, blk < NBL2)
            own1 = jnp.logical_and(cid == 1, blk >= NBL2)

            @pl.when(jnp.logical_or(own0, own1))
            def _():
                pltpu.sync_copy(av_b, alpha_hbm.at[pl.ds(eoff, BL)])
            return c
        lax.fori_loop(0, NBL, _p1blk, 0)
        plsc.subcore_barrier()
        sc1.__exit__(None, None, None)

        sc2 = jax.named_scope("p15_dinv")
        sc2.__enter__()
        th = jnp.full((G,), 1.5, _F32)
        hf = jnp.full((G,), 0.5, _F32)
        magic = jnp.full((G,), 0x5F3759DF, _I32)

        def _dk(k, c):
            doff = sid * NCHUNK + k * SB
            pltpu.sync_copy(deg_sh.at[pl.ds(doff, SB)], dchunk)
            for g in range(GB):
                d = dchunk[pl.ds(g * G, G)]
                y = plsc.bitcast(magic - (plsc.bitcast(d, _I32) >> 1), _F32)
                hd = hf * d
                y = y * (th - hd * y * y)
                y = y * (th - hd * y * y)
                y = y * (th - hd * y * y)
                dchunk[pl.ds(g * G, G)] = jnp.where(d > 0.0, y, zv)
            pltpu.sync_copy(dchunk, deg_sh.at[pl.ds(doff, SB)])
            return c
        lax.fori_loop(0, NCHUNK // SB, _dk, 0)
        plsc.subcore_barrier()
        pltpu.sync_copy(deg_sh, tbl_a)
        sc2.__exit__(None, None, None)

        sc3 = jax.named_scope("p2_aggregate")
        sc3.__enter__()
        def _stage2(j, ridx, cidx, nrm):
            for gg in range(GB):
                boff = j * SB + gg * G
                r = rows_b[pl.ds(boff, G)]
                cc = cols_b[pl.ds(boff, G)]
                a = av_b[pl.ds(boff, G)]
                d1 = plsc.load_gather(tbl_a, [r])
                d2 = plsc.load_gather(tbl_a, [cc])
                ridx[0, pl.ds(gg * G, G)] = r
                cidx[0, pl.ds(gg * G, G)] = cc
                nrm[pl.ds(gg * G, G)] = d1 * a * d2

        def _scale(rowbuf, nrm):
            def _sg(g, c):
                nvec = nrm[pl.ds(g * G, G)]
                for e in range(G):
                    nb = nvec.at[jnp.full((G,), e, _I32)].get(
                        mode="promise_in_bounds")
                    eidx = g * G + e
                    for jj in range(CW):
                        rowbuf[eidx, pl.ds(jj * G, G)] = (
                            rowbuf[eidx, pl.ds(jj * G, G)] * nb)
                return c
            lax.fori_loop(0, GB, _sg, 0)

        def _p2blk(blk, c):
            eoff = ebase + cid * EP2 + blk * BL
            pltpu.sync_copy(row_hbm.at[pl.ds(eoff, BL)], rows_b)
            pltpu.sync_copy(col_hbm.at[pl.ds(eoff, BL)], cols_b)
            pltpu.sync_copy(alpha_hbm.at[pl.ds(eoff, BL)], av_b)

            def _pipe2(t, c2):
                for p in range(2):
                    ridx, cidx, _, nrm, rowbuf, gsem, ssem = slots[p]

                    @pl.when(t > 0)
                    def _():
                        pltpu.make_async_copy(
                            rowbuf, acc_sh.at[ridx.at[0]], ssem).wait()
                    _stage2(2 * t + p, ridx, cidx, nrm)
                    pltpu.async_copy(h_hbm.at[cidx.at[0]], rowbuf, gsem)
                for p in range(2):
                    ridx, cidx, _, nrm, rowbuf, gsem, ssem = slots[p]
                    pltpu.make_async_copy(h_hbm.at[cidx.at[0]], rowbuf,
                                          gsem).wait()
                    _scale(rowbuf, nrm)
                    pltpu.async_copy(rowbuf, acc_sh.at[ridx.at[0]],
                                     ssem, add=True)
                return c2
            lax.fori_loop(0, NPIPE, _pipe2, 0)
            for p in range(2):
                ridx, _, _, _, rowbuf, _, ssem = slots[p]
                pltpu.make_async_copy(rowbuf, acc_sh.at[ridx.at[0]],
                                      ssem).wait()
            _stage2(NSB - 1, ridx0, cidx0, nrm0)
            pltpu.sync_copy(h_hbm.at[cidx0.at[0]], rowbuf0)
            _scale(rowbuf0, nrm0)
            pltpu.sync_copy(rowbuf0, acc_sh.at[ridx0.at[0]], add=True)
            return c
        lax.fori_loop(0, NBL2, _p2blk, 0)
        plsc.subcore_barrier()
        sc3.__exit__(None, None, None)

        for k in range(NCHUNK // SB):
            roff = sid * NCHUNK + k * SB
            pltpu.sync_copy(acc_sh.at[pl.ds(roff, SB)], rowbuf0)
            pltpu.sync_copy(rowbuf0, part_hbm.at[cid, pl.ds(roff, SB)])

    return pl.kernel(
        body,
        out_type=(jax.ShapeDtypeStruct((E,), _F32),
                  jax.ShapeDtypeStruct((NC, Npad, C), _F32)),
        mesh=mesh,
        compiler_params=pltpu.CompilerParams(needs_layout_passes=False,
                                             use_tc_tiling_on_sc=False),
        scratch_types=[
            pltpu.VMEM((Npad,), _F32),
            pltpu.VMEM((N,), _F32),
            pltpu.VMEM((BL,), _I32),
            pltpu.VMEM((BL,), _I32),
            pltpu.VMEM((BL,), _F32),
            pltpu.VMEM((SB,), _F32),
            pltpu.VMEM((SB,), _F32),
            pltpu.VMEM((SB,), _F32),
            pltpu.VMEM((1, SB), _I32),
            pltpu.VMEM((1, SB), _I32),
            pltpu.VMEM((1, SB), _I32),
            pltpu.VMEM((1, SB), _I32),
            pltpu.VMEM((1, SB), _F32),
            pltpu.VMEM((1, SB), _F32),
            pltpu.VMEM((1, SB), _F32),
            pltpu.VMEM((1, SB), _F32),
            pltpu.VMEM((SB, C), _F32),
            pltpu.VMEM((SB, C), _F32),
            pltpu.SemaphoreType.DMA,
            pltpu.SemaphoreType.DMA,
            pltpu.SemaphoreType.DMA,
            pltpu.SemaphoreType.DMA,
            pltpu.VMEM_SHARED((Npad,), _F32),
            pltpu.VMEM_SHARED((Npad, C), _F32),
        ],
    )


def kernel(x, edge_index, edge_attr, batch_mask, weight, alpha_fc_w, alpha_fc_b):
    N, IC = x.shape
    OC = weight.shape[1]
    E = edge_index.shape[1]
    Npad = ((N + 1279) // 1280) * 1280

    row = edge_index[0].astype(_I32)
    col = edge_index[1].astype(_I32)
    attr = edge_attr.astype(_F32)

    wpair = alpha_fc_w[0].reshape(2, OC)
    bpair = jnp.zeros((2, 1), _F32).at[0, 0].set(alpha_fc_b[0])

    h, s = pl.pallas_call(
        _front_body,
        out_shape=(jax.ShapeDtypeStruct((N, OC), _F32),
                   jax.ShapeDtypeStruct((2, N), _F32)),
    )(x, weight, wpair, bpair)
    s1 = s[0]
    s2 = s[1]

    sc_edge = _make_sc_edge(N, Npad, E, OC)
    alpha, partials = sc_edge(row, col, attr, s1, s2, h)

    CH = N // 5
    out2 = pl.pallas_call(
        _add_body,
        grid=(5,),
        in_specs=[pl.BlockSpec((2, CH, OC), lambda i: (0, i, 0))],
        out_specs=pl.BlockSpec((CH, OC), lambda i: (i, 0)),
        out_shape=jax.ShapeDtypeStruct((N, OC), _F32),
    )(partials)

    out = out2.reshape(N, OC, 1)
    return (out, alpha.reshape(E, 1), edge_index)
